# Initial kernel scaffold; baseline (speedup 1.0000x reference)
import jax
import jax.numpy as jnp
from jax.experimental import pallas as pl


def kernel(ref_feats, tgt_feats, ref_mask, original_size):
    raise NotImplementedError("write your pallas kernel here")


# speedup vs baseline: 1.6838x; 1.6838x over previous
import functools

import jax
import jax.numpy as jnp
from jax import lax
from jax.experimental import pallas as pl
from jax.experimental.pallas import tpu as pltpu
from jax.experimental.pallas import tpu_sc as plsc

N = 4096
D = 768
BR = 512
NBLK = N // BR
NEG = -1e9
FEAT = 64
FEAT_SHIFT = 6
PATCH = 16
INPUT_SIZE = 1024
NUM_FG = 40
NUM_BG = 2
MAX_POINTS = 64


MATMUL_MODE = "bf16"


def _stage1_body(ref_ref, mask_ref, tgtt_ref,
                 fwidx_ref, fwsc_ref, okt_ref, csum_ref,
                 tnh_ref, tnl_ref, cmax_all_ref, cmax_m_ref, csum_acc_ref):
    i = pl.program_id(0)

    @pl.when(i == 0)
    def _init():
        t = tgtt_ref[...]
        nrm = jnp.sqrt(jnp.sum(t * t, axis=0, keepdims=True))
        tn = t / (nrm + 1e-6)
        tnh = tn.astype(jnp.bfloat16)
        tnh_ref[...] = tnh
        if MATMUL_MODE == "bf16x3":
            tnl_ref[...] = (tn - tnh.astype(jnp.float32)).astype(jnp.bfloat16)
        elif MATMUL_MODE == "f32":
            tnl_ref[...] = tn
        cmax_all_ref[...] = jnp.full((1, N), -jnp.inf, jnp.float32)
        cmax_m_ref[...] = jnp.full((1, N), -jnp.inf, jnp.float32)
        csum_acc_ref[...] = jnp.zeros((1, N), jnp.float32)

    r = ref_ref[...]
    rn = r / (jnp.sqrt(jnp.sum(r * r, axis=1, keepdims=True)) + 1e-6)
    dotf = functools.partial(jnp.dot, preferred_element_type=jnp.float32)
    if MATMUL_MODE == "bf16":
        sim = dotf(rn.astype(jnp.bfloat16), tnh_ref[...])
    elif MATMUL_MODE == "bf16x3":
        rh = rn.astype(jnp.bfloat16)
        rl = (rn - rh.astype(jnp.float32)).astype(jnp.bfloat16)
        sim = (dotf(rh, tnl_ref[...]) + dotf(rl, tnh_ref[...])
               + dotf(rh, tnh_ref[...]))
    else:
        sim = dotf(rn, tnl_ref[...])

    rowmax = jnp.max(sim, axis=1)
    colio = lax.broadcasted_iota(jnp.int32, (BR, N), 1)
    fwidx = jnp.min(jnp.where(sim == rowmax[:, None], colio, N), axis=1)
    fwidx_ref[0, 0, :] = fwidx
    fwsc_ref[0, 0, :] = rowmax

    m = mask_ref[0, 0, :]
    mcol = m[:, None] > 0
    bmax_all = jnp.max(sim, axis=0)
    bmax_m = jnp.max(jnp.where(mcol, sim, -jnp.inf), axis=0)
    bsum = jnp.sum(jnp.where(mcol, sim, 0.0), axis=0)
    cmax_all_ref[...] = jnp.maximum(cmax_all_ref[...], bmax_all[None, :])
    cmax_m_ref[...] = jnp.maximum(cmax_m_ref[...], bmax_m[None, :])
    csum_acc_ref[...] = csum_acc_ref[...] + bsum[None, :]

    @pl.when(i == NBLK - 1)
    def _fin():
        okt_ref[...] = (cmax_m_ref[...] == cmax_all_ref[...]).astype(jnp.int32)
        csum_ref[...] = csum_acc_ref[...]


def _stage1(ref_feats, mask3, tgt_t):
    return pl.pallas_call(
        _stage1_body,
        grid=(NBLK,),
        in_specs=[
            pl.BlockSpec((BR, D), lambda i: (i, 0)),
            pl.BlockSpec((1, 1, BR), lambda i: (i, 0, 0)),
            pl.BlockSpec((D, N), lambda i: (0, 0)),
        ],
        out_specs=[
            pl.BlockSpec((1, 1, BR), lambda i: (i, 0, 0)),
            pl.BlockSpec((1, 1, BR), lambda i: (i, 0, 0)),
            pl.BlockSpec((1, N), lambda i: (0, 0)),
            pl.BlockSpec((1, N), lambda i: (0, 0)),
        ],
        out_shape=[
            jax.ShapeDtypeStruct((NBLK, 1, BR), jnp.int32),
            jax.ShapeDtypeStruct((NBLK, 1, BR), jnp.float32),
            jax.ShapeDtypeStruct((1, N), jnp.int32),
            jax.ShapeDtypeStruct((1, N), jnp.float32),
        ],
        scratch_shapes=[
            pltpu.VMEM((D, N), jnp.bfloat16),
            (pltpu.VMEM((D, N), jnp.bfloat16) if MATMUL_MODE == "bf16x3"
             else pltpu.VMEM((8, 128), jnp.float32) if MATMUL_MODE == "bf16"
             else pltpu.VMEM((D, N), jnp.float32)),
            pltpu.VMEM((1, N), jnp.float32),
            pltpu.VMEM((1, N), jnp.float32),
            pltpu.VMEM((1, N), jnp.float32),
        ],
        compiler_params=pltpu.CompilerParams(
            dimension_semantics=("arbitrary",),
        ),
    )(ref_feats, mask3, tgt_t)


NCH = N // 16


def _round_even(v):
    y = v + 0.5
    t = y.astype(jnp.int32)
    exact = t.astype(jnp.float32) == y
    odd = (t & 1) == 1
    t = jnp.where(exact & odd, t - 1, t)
    return t.astype(jnp.float32)


def _sc_body(fwt_hbm, fws_hbm, okt_hbm, csum_hbm, mask_hbm, osz_hbm, out_hbm,
             fwt_v, fws_v, okt_v, csum_v, mask_v, osz_v, sc2_v, cm_v, out_v):
    cid = lax.axis_index("c")
    sid = lax.axis_index("s")
    wid = sid * 2 + cid

    @pl.when(wid == 0)
    def _():
        pltpu.sync_copy(fwt_hbm, fwt_v)
        pltpu.sync_copy(fws_hbm, fws_v)
        pltpu.sync_copy(okt_hbm, okt_v)
        pltpu.sync_copy(csum_hbm, csum_v)
        pltpu.sync_copy(mask_hbm, mask_v)
        pltpu.sync_copy(osz_hbm, osz_v)

        io16 = lax.iota(jnp.int32, 16)
        negv = jnp.full((16,), NEG, jnp.float32)
        neg1 = jnp.full((16,), -1.0, jnp.float32)
        lowv = jnp.full((16,), -3e38, jnp.float32)

        ov = osz_v[pl.ds(0, 16)]
        zi = jnp.zeros((16,), jnp.int32)
        h_sc = jnp.max(jnp.where(io16 == 0, ov, zi))
        w_sc = jnp.max(jnp.where(io16 == 1, ov, zi))
        shv = jnp.full((16,), h_sc).astype(jnp.float32) / float(INPUT_SIZE)
        swv = jnp.full((16,), w_sc).astype(jnp.float32) / float(INPUT_SIZE)

        def _dsum(j, acc):
            return acc + mask_v[pl.ds(j * 16, 16)]
        msum = lax.fori_loop(0, NCH, _dsum, jnp.zeros((16,), jnp.int32))
        denv = jnp.maximum(jnp.full((16,), jnp.sum(msum)).astype(jnp.float32),
                           1.0)

        def _oinit(j, c):
            out_v[pl.ds(j * 16, 16)] = neg1
            return c
        lax.fori_loop(0, (MAX_POINTS * 4) // 16, _oinit, 0)

        def _build(j, c):
            tv = fwt_v[pl.ds(j * 16, 16)]
            okv = plsc.load_gather(okt_v, [tv])
            mv = mask_v[pl.ds(j * 16, 16)]
            sv = fws_v[pl.ds(j * 16, 16)]
            sc = jnp.where((mv > 0) & (okv > 0), sv, negv)
            jv = jnp.full((16,), j, jnp.int32)
            plsc.store_scatter(sc2_v, [jv, io16], sc)
            plsc.store_scatter(cm_v, [jv], jnp.full((16,), jnp.max(sc)))
            return c
        lax.fori_loop(0, NCH, _build, 0)

        def _extract():
            def _gm(j, mx):
                return jnp.maximum(mx, cm_v[pl.ds(j * 16, 16)])
            mx = lax.fori_loop(0, NCH // 16, _gm, lowv)
            gmax = jnp.max(mx)
            gs = jnp.full((16,), gmax)

            def _pc(j, pm):
                v = cm_v[pl.ds(j * 16, 16)]
                pos = io16 + j * 16
                return jnp.minimum(pm, jnp.where(v == gs, pos, N))
            pmv = lax.fori_loop(0, NCH // 16, _pc,
                                jnp.full((16,), N, jnp.int32))
            c = jnp.min(pmv)
            cv = jnp.full((16,), c, jnp.int32)
            row = plsc.load_gather(sc2_v, [cv, io16])
            l = jnp.min(jnp.where(row == gs, io16, 16))
            idx = c * 16 + l
            nrow = jnp.where(io16 == l, lowv, row)
            plsc.store_scatter(sc2_v, [cv, io16], nrow)
            plsc.store_scatter(cm_v, [cv], jnp.full((16,), jnp.max(nrow)))
            return idx, gs

        def _pack(m, xi, yi, scv, labv, acc):
            acc = jnp.where(io16 == 4 * m, xi, acc)
            acc = jnp.where(io16 == 4 * m + 1, yi, acc)
            acc = jnp.where(io16 == 4 * m + 2, scv, acc)
            acc = jnp.where(io16 == 4 * m + 3, labv, acc)
            return acc

        def _fg(k, acc):
            idx, gs = _extract()
            tgt = plsc.load_gather(fwt_v, [jnp.full((16,), idx, jnp.int32)])
            okp = gs > (NEG / 2)
            scv = jnp.where(okp, gs, -1.0)
            labv = jnp.where(okp, 1.0, -1.0)
            xf = (tgt & (FEAT - 1)).astype(jnp.float32) * float(PATCH) + float(PATCH // 2)
            yf = lax.shift_right_logical(tgt, FEAT_SHIFT).astype(jnp.float32) * float(PATCH) + float(PATCH // 2)
            xi = jnp.where(okp, _round_even(xf * swv), -1.0)
            yi = jnp.where(okp, _round_even(yf * shv), -1.0)
            m = lax.rem(k, 4)
            acc = _pack(m, xi, yi, scv, labv, acc)

            @pl.when(m == 3)
            def _st():
                out_v[pl.ds((k // 4) * 16, 16)] = acc
            return jnp.where(jnp.full((16,), m, jnp.int32) == 3, neg1, acc)
        lax.fori_loop(0, NUM_FG, _fg, neg1)

        def _bbuild(j, c):
            sc = -(csum_v[pl.ds(j * 16, 16)] / denv)
            jv = jnp.full((16,), j, jnp.int32)
            plsc.store_scatter(sc2_v, [jv, io16], sc)
            plsc.store_scatter(cm_v, [jv], jnp.full((16,), jnp.max(sc)))
            return c
        lax.fori_loop(0, NCH, _bbuild, 0)

        def _bg(k, acc):
            idx, gs = _extract()
            tgt = jnp.full((16,), idx, jnp.int32)
            xf = (tgt & (FEAT - 1)).astype(jnp.float32) * float(PATCH) + float(PATCH // 2)
            yf = lax.shift_right_logical(tgt, FEAT_SHIFT).astype(jnp.float32) * float(PATCH) + float(PATCH // 2)
            xi = _round_even(xf * swv)
            yi = _round_even(yf * shv)
            return _pack(k, xi, yi, -gs, jnp.zeros((16,), jnp.float32), acc)
        acc2 = lax.fori_loop(0, NUM_BG, _bg, neg1)
        out_v[pl.ds((NUM_FG // 4) * 16, 16)] = acc2

        pltpu.sync_copy(out_v, out_hbm)


def _stage2(fwt, fws, okt, csum, mask_i, osz16):
    mesh = plsc.VectorSubcoreMesh(core_axis_name="c", subcore_axis_name="s")
    f = functools.partial(
        pl.kernel,
        mesh=mesh,
        out_type=jax.ShapeDtypeStruct((MAX_POINTS * 4,), jnp.float32),
        scratch_types=[
            pltpu.VMEM((N,), jnp.int32),
            pltpu.VMEM((N,), jnp.float32),
            pltpu.VMEM((N,), jnp.int32),
            pltpu.VMEM((N,), jnp.float32),
            pltpu.VMEM((N,), jnp.int32),
            pltpu.VMEM((16,), jnp.int32),
            pltpu.VMEM((NCH, 16), jnp.float32),
            pltpu.VMEM((NCH,), jnp.float32),
            pltpu.VMEM((MAX_POINTS * 4,), jnp.float32),
        ],
        compiler_params=pltpu.CompilerParams(needs_layout_passes=False),
    )(_sc_body)
    return f(fwt, fws, okt, csum, mask_i, osz16)


def kernel(ref_feats, tgt_feats, ref_mask, original_size):
    mask_i = ref_mask.astype(jnp.int32)
    mask3 = mask_i.reshape(NBLK, 1, BR)
    tgt_t = tgt_feats.T
    fwidx, fwsc, okt, csum = _stage1(ref_feats, mask3, tgt_t)
    osz16 = jnp.zeros((16,), jnp.int32).at[:2].set(original_size)
    out = _stage2(fwidx.reshape(N), fwsc.reshape(N), okt.reshape(N),
                  csum.reshape(N), mask_i, osz16)
    return out.reshape(MAX_POINTS, 4)


# speedup vs baseline: 2.0850x; 1.2383x over previous
import functools

import jax
import jax.numpy as jnp
from jax import lax
from jax.experimental import pallas as pl
from jax.experimental.pallas import tpu as pltpu
from jax.experimental.pallas import tpu_sc as plsc

N = 4096
D = 768
BR = 512
NBLK = N // BR
NEG = -1e9
FEAT = 64
FEAT_SHIFT = 6
PATCH = 16
INPUT_SIZE = 1024
NUM_FG = 40
NUM_BG = 2
MAX_POINTS = 64


MATMUL_MODE = "bf16"


def _stage1_body(ref_ref, mask_ref, tgtt_ref,
                 fwidx_ref, fwsc_ref, okt_ref, csum_ref,
                 tnh_ref, tnl_ref, cmax_all_ref, cmax_m_ref, csum_acc_ref):
    i = pl.program_id(0)

    @pl.when(i == 0)
    def _init():
        t = tgtt_ref[...]
        nrm = jnp.sqrt(jnp.sum(t * t, axis=1, keepdims=True))
        tn = t / (nrm + 1e-6)
        tnh = tn.astype(jnp.bfloat16)
        tnh_ref[...] = tnh
        if MATMUL_MODE == "bf16x3":
            tnl_ref[...] = (tn - tnh.astype(jnp.float32)).astype(jnp.bfloat16)
        elif MATMUL_MODE == "f32":
            tnl_ref[...] = tn
        cmax_all_ref[...] = jnp.full((1, N), -jnp.inf, jnp.float32)
        cmax_m_ref[...] = jnp.full((1, N), -jnp.inf, jnp.float32)
        csum_acc_ref[...] = jnp.zeros((1, N), jnp.float32)

    r = ref_ref[...]
    rn = r / (jnp.sqrt(jnp.sum(r * r, axis=1, keepdims=True)) + 1e-6)
    dotf = functools.partial(jnp.dot, preferred_element_type=jnp.float32)
    if MATMUL_MODE == "bf16":
        sim = lax.dot_general(rn.astype(jnp.bfloat16), tnh_ref[...],
                              dimension_numbers=(((1,), (1,)), ((), ())),
                              preferred_element_type=jnp.float32)
    elif MATMUL_MODE == "bf16x3":
        rh = rn.astype(jnp.bfloat16)
        rl = (rn - rh.astype(jnp.float32)).astype(jnp.bfloat16)
        sim = (dotf(rh, tnl_ref[...]) + dotf(rl, tnh_ref[...])
               + dotf(rh, tnh_ref[...]))
    else:
        sim = dotf(rn, tnl_ref[...])

    rowmax = jnp.max(sim, axis=1)
    colio = lax.broadcasted_iota(jnp.int32, (BR, N), 1)
    fwidx = jnp.min(jnp.where(sim == rowmax[:, None], colio, N), axis=1)
    fwidx_ref[0, 0, :] = fwidx
    fwsc_ref[0, 0, :] = rowmax

    m = mask_ref[0, 0, :]
    mcol = m[:, None] > 0
    bmax_all = jnp.max(sim, axis=0)
    bmax_m = jnp.max(jnp.where(mcol, sim, -jnp.inf), axis=0)
    bsum = jnp.sum(jnp.where(mcol, sim, 0.0), axis=0)
    cmax_all_ref[...] = jnp.maximum(cmax_all_ref[...], bmax_all[None, :])
    cmax_m_ref[...] = jnp.maximum(cmax_m_ref[...], bmax_m[None, :])
    csum_acc_ref[...] = csum_acc_ref[...] + bsum[None, :]

    @pl.when(i == NBLK - 1)
    def _fin():
        okt_ref[...] = (cmax_m_ref[...] == cmax_all_ref[...]).astype(jnp.int32)
        csum_ref[...] = csum_acc_ref[...]


def _stage1(ref_feats, mask3, tgt_t):
    return pl.pallas_call(
        _stage1_body,
        grid=(NBLK,),
        in_specs=[
            pl.BlockSpec((BR, D), lambda i: (i, 0)),
            pl.BlockSpec((1, 1, BR), lambda i: (i, 0, 0)),
            pl.BlockSpec((N, D), lambda i: (0, 0)),
        ],
        out_specs=[
            pl.BlockSpec((1, 1, BR), lambda i: (i, 0, 0)),
            pl.BlockSpec((1, 1, BR), lambda i: (i, 0, 0)),
            pl.BlockSpec((1, N), lambda i: (0, 0)),
            pl.BlockSpec((1, N), lambda i: (0, 0)),
        ],
        out_shape=[
            jax.ShapeDtypeStruct((NBLK, 1, BR), jnp.int32),
            jax.ShapeDtypeStruct((NBLK, 1, BR), jnp.float32),
            jax.ShapeDtypeStruct((1, N), jnp.int32),
            jax.ShapeDtypeStruct((1, N), jnp.float32),
        ],
        scratch_shapes=[
            pltpu.VMEM((N, D), jnp.bfloat16),
            (pltpu.VMEM((D, N), jnp.bfloat16) if MATMUL_MODE == "bf16x3"
             else pltpu.VMEM((8, 128), jnp.float32) if MATMUL_MODE == "bf16"
             else pltpu.VMEM((D, N), jnp.float32)),
            pltpu.VMEM((1, N), jnp.float32),
            pltpu.VMEM((1, N), jnp.float32),
            pltpu.VMEM((1, N), jnp.float32),
        ],
        compiler_params=pltpu.CompilerParams(
            dimension_semantics=("arbitrary",),
        ),
    )(ref_feats, mask3, tgt_t)


NCH = N // 16


def _round_even(v):
    y = v + 0.5
    t = y.astype(jnp.int32)
    exact = t.astype(jnp.float32) == y
    odd = (t & 1) == 1
    t = jnp.where(exact & odd, t - 1, t)
    return t.astype(jnp.float32)


def _sc_body(fwt_hbm, fws_hbm, okt_hbm, csum_hbm, mask_hbm, osz_hbm, out_hbm,
             fwt_v, fws_v, okt_v, csum_v, mask_v, osz_v, sc2_v, cm_v, out_v):
    cid = lax.axis_index("c")
    sid = lax.axis_index("s")
    wid = sid * 2 + cid

    @pl.when(wid == 0)
    def _():
        pltpu.sync_copy(fwt_hbm, fwt_v)
        pltpu.sync_copy(fws_hbm, fws_v)
        pltpu.sync_copy(okt_hbm, okt_v)
        pltpu.sync_copy(csum_hbm, csum_v)
        pltpu.sync_copy(mask_hbm, mask_v)
        pltpu.sync_copy(osz_hbm, osz_v)

        io16 = lax.iota(jnp.int32, 16)
        negv = jnp.full((16,), NEG, jnp.float32)
        neg1 = jnp.full((16,), -1.0, jnp.float32)
        lowv = jnp.full((16,), -3e38, jnp.float32)

        ov = osz_v[pl.ds(0, 16)]
        zi = jnp.zeros((16,), jnp.int32)
        h_sc = jnp.max(jnp.where(io16 == 0, ov, zi))
        w_sc = jnp.max(jnp.where(io16 == 1, ov, zi))
        shv = jnp.full((16,), h_sc).astype(jnp.float32) / float(INPUT_SIZE)
        swv = jnp.full((16,), w_sc).astype(jnp.float32) / float(INPUT_SIZE)

        def _dsum(j, acc):
            return acc + mask_v[pl.ds(j * 16, 16)]
        msum = lax.fori_loop(0, NCH, _dsum, jnp.zeros((16,), jnp.int32))
        denv = jnp.maximum(jnp.full((16,), jnp.sum(msum)).astype(jnp.float32),
                           1.0)

        def _oinit(j, c):
            out_v[pl.ds(j * 16, 16)] = neg1
            return c
        lax.fori_loop(0, (MAX_POINTS * 4) // 16, _oinit, 0)

        def _build(j, c):
            tv = fwt_v[pl.ds(j * 16, 16)]
            okv = plsc.load_gather(okt_v, [tv])
            mv = mask_v[pl.ds(j * 16, 16)]
            sv = fws_v[pl.ds(j * 16, 16)]
            sc = jnp.where((mv > 0) & (okv > 0), sv, negv)
            jv = jnp.full((16,), j, jnp.int32)
            plsc.store_scatter(sc2_v, [jv, io16], sc)
            plsc.store_scatter(cm_v, [jv], jnp.full((16,), jnp.max(sc)))
            return c
        lax.fori_loop(0, NCH, _build, 0)

        def _sup16():
            cm2 = lowv
            for s in range(16):
                ms = jnp.max(cm_v[pl.ds(s * 16, 16)])
                cm2 = jnp.where(io16 == s, jnp.full((16,), ms), cm2)
            return cm2

        def _extract(cm2):
            gmax = jnp.max(cm2)
            gs = jnp.full((16,), gmax)
            s = jnp.min(jnp.where(cm2 == gs, io16, 16))
            cmv = cm_v[pl.ds(s * 16, 16)]
            c16 = jnp.min(jnp.where(cmv == gs, io16, 16))
            c = s * 16 + c16
            cv = jnp.full((16,), c, jnp.int32)
            row = plsc.load_gather(sc2_v, [cv, io16])
            l = jnp.min(jnp.where(row == gs, io16, 16))
            idx = c * 16 + l
            nrow = jnp.where(io16 == l, lowv, row)
            plsc.store_scatter(sc2_v, [cv, io16], nrow)
            newc = jnp.full((16,), jnp.max(nrow))
            plsc.store_scatter(cm_v, [cv], newc)
            cmv2 = jnp.where(io16 == c16, newc, cmv)
            cm2 = jnp.where(io16 == s, jnp.full((16,), jnp.max(cmv2)), cm2)
            return idx, gs, cm2

        def _pack(m, xi, yi, scv, labv, acc):
            acc = jnp.where(io16 == 4 * m, xi, acc)
            acc = jnp.where(io16 == 4 * m + 1, yi, acc)
            acc = jnp.where(io16 == 4 * m + 2, scv, acc)
            acc = jnp.where(io16 == 4 * m + 3, labv, acc)
            return acc

        def _fg(k, carry):
            acc, cm2 = carry
            idx, gs, cm2 = _extract(cm2)
            tgt = plsc.load_gather(fwt_v, [jnp.full((16,), idx, jnp.int32)])
            okp = gs > (NEG / 2)
            scv = jnp.where(okp, gs, -1.0)
            labv = jnp.where(okp, 1.0, -1.0)
            xf = (tgt & (FEAT - 1)).astype(jnp.float32) * float(PATCH) + float(PATCH // 2)
            yf = lax.shift_right_logical(tgt, FEAT_SHIFT).astype(jnp.float32) * float(PATCH) + float(PATCH // 2)
            xi = jnp.where(okp, _round_even(xf * swv), -1.0)
            yi = jnp.where(okp, _round_even(yf * shv), -1.0)
            m = lax.rem(k, 4)
            acc = _pack(m, xi, yi, scv, labv, acc)

            @pl.when(m == 3)
            def _st():
                out_v[pl.ds((k // 4) * 16, 16)] = acc
            acc = jnp.where(jnp.full((16,), m, jnp.int32) == 3, neg1, acc)
            return acc, cm2
        lax.fori_loop(0, NUM_FG, _fg, (neg1, _sup16()))

        def _bbuild(j, c):
            sc = -(csum_v[pl.ds(j * 16, 16)] / denv)
            jv = jnp.full((16,), j, jnp.int32)
            plsc.store_scatter(sc2_v, [jv, io16], sc)
            plsc.store_scatter(cm_v, [jv], jnp.full((16,), jnp.max(sc)))
            return c
        lax.fori_loop(0, NCH, _bbuild, 0)

        def _bg(k, carry):
            acc, cm2 = carry
            idx, gs, cm2 = _extract(cm2)
            tgt = jnp.full((16,), idx, jnp.int32)
            xf = (tgt & (FEAT - 1)).astype(jnp.float32) * float(PATCH) + float(PATCH // 2)
            yf = lax.shift_right_logical(tgt, FEAT_SHIFT).astype(jnp.float32) * float(PATCH) + float(PATCH // 2)
            xi = _round_even(xf * swv)
            yi = _round_even(yf * shv)
            return _pack(k, xi, yi, -gs, jnp.zeros((16,), jnp.float32), acc), cm2
        acc2, _ = lax.fori_loop(0, NUM_BG, _bg, (neg1, _sup16()))
        out_v[pl.ds((NUM_FG // 4) * 16, 16)] = acc2

        pltpu.sync_copy(out_v, out_hbm)


def _stage2(fwt, fws, okt, csum, mask_i, osz16):
    mesh = plsc.VectorSubcoreMesh(core_axis_name="c", subcore_axis_name="s")
    f = functools.partial(
        pl.kernel,
        mesh=mesh,
        out_type=jax.ShapeDtypeStruct((MAX_POINTS * 4,), jnp.float32),
        scratch_types=[
            pltpu.VMEM((N,), jnp.int32),
            pltpu.VMEM((N,), jnp.float32),
            pltpu.VMEM((N,), jnp.int32),
            pltpu.VMEM((N,), jnp.float32),
            pltpu.VMEM((N,), jnp.int32),
            pltpu.VMEM((16,), jnp.int32),
            pltpu.VMEM((NCH, 16), jnp.float32),
            pltpu.VMEM((NCH,), jnp.float32),
            pltpu.VMEM((MAX_POINTS * 4,), jnp.float32),
        ],
        compiler_params=pltpu.CompilerParams(needs_layout_passes=False),
    )(_sc_body)
    return f(fwt, fws, okt, csum, mask_i, osz16)


def kernel(ref_feats, tgt_feats, ref_mask, original_size):
    mask_i = ref_mask.astype(jnp.int32)
    mask3 = mask_i.reshape(NBLK, 1, BR)
    tgt_t = tgt_feats
    fwidx, fwsc, okt, csum = _stage1(ref_feats, mask3, tgt_t)
    osz16 = jnp.zeros((16,), jnp.int32).at[:2].set(original_size)
    out = _stage2(fwidx.reshape(N), fwsc.reshape(N), okt.reshape(N),
                  csum.reshape(N), mask_i, osz16)
    return out.reshape(MAX_POINTS, 4)


# speedup vs baseline: 2.1462x; 1.0294x over previous
import functools

import jax
import jax.numpy as jnp
from jax import lax
from jax.experimental import pallas as pl
from jax.experimental.pallas import tpu as pltpu
from jax.experimental.pallas import tpu_sc as plsc

N = 4096
D = 768
BR = 1024
NBLK = N // BR
NEG = -1e9
FEAT = 64
FEAT_SHIFT = 6
PATCH = 16
INPUT_SIZE = 1024
NUM_FG = 40
NUM_BG = 2
MAX_POINTS = 64


MATMUL_MODE = "bf16"


def _stage1_body(ref_ref, mask_ref, tgtt_ref,
                 fwidx_ref, fwsc_ref, okt_ref, csum_ref,
                 tnh_ref, tnl_ref, cmax_all_ref, cmax_m_ref, csum_acc_ref):
    i = pl.program_id(0)

    @pl.when(i == 0)
    def _init():
        t = tgtt_ref[...]
        nrm = jnp.sqrt(jnp.sum(t * t, axis=1, keepdims=True))
        tn = t / (nrm + 1e-6)
        tnh = tn.astype(jnp.bfloat16)
        tnh_ref[...] = tnh
        if MATMUL_MODE == "bf16x3":
            tnl_ref[...] = (tn - tnh.astype(jnp.float32)).astype(jnp.bfloat16)
        elif MATMUL_MODE == "f32":
            tnl_ref[...] = tn
        cmax_all_ref[...] = jnp.full((1, N), -jnp.inf, jnp.float32)
        cmax_m_ref[...] = jnp.full((1, N), -jnp.inf, jnp.float32)
        csum_acc_ref[...] = jnp.zeros((1, N), jnp.float32)

    r = ref_ref[...]
    rn = r / (jnp.sqrt(jnp.sum(r * r, axis=1, keepdims=True)) + 1e-6)
    dotf = functools.partial(jnp.dot, preferred_element_type=jnp.float32)
    if MATMUL_MODE == "bf16":
        sim = lax.dot_general(rn.astype(jnp.bfloat16), tnh_ref[...],
                              dimension_numbers=(((1,), (1,)), ((), ())),
                              preferred_element_type=jnp.float32)
    elif MATMUL_MODE == "bf16x3":
        rh = rn.astype(jnp.bfloat16)
        rl = (rn - rh.astype(jnp.float32)).astype(jnp.bfloat16)
        sim = (dotf(rh, tnl_ref[...]) + dotf(rl, tnh_ref[...])
               + dotf(rh, tnh_ref[...]))
    else:
        sim = dotf(rn, tnl_ref[...])

    rowmax = jnp.max(sim, axis=1)
    colio = lax.broadcasted_iota(jnp.int32, (BR, N), 1)
    fwidx = jnp.min(jnp.where(sim == rowmax[:, None], colio, N), axis=1)
    fwidx_ref[0, 0, :] = fwidx
    fwsc_ref[0, 0, :] = rowmax

    m = mask_ref[0, 0, :]
    mcol = m[:, None] > 0
    bmax_all = jnp.max(sim, axis=0)
    bmax_m = jnp.max(jnp.where(mcol, sim, -jnp.inf), axis=0)
    bsum = jnp.sum(jnp.where(mcol, sim, 0.0), axis=0)
    cmax_all_ref[...] = jnp.maximum(cmax_all_ref[...], bmax_all[None, :])
    cmax_m_ref[...] = jnp.maximum(cmax_m_ref[...], bmax_m[None, :])
    csum_acc_ref[...] = csum_acc_ref[...] + bsum[None, :]

    @pl.when(i == NBLK - 1)
    def _fin():
        okt_ref[...] = (cmax_m_ref[...] == cmax_all_ref[...]).astype(jnp.int32)
        csum_ref[...] = csum_acc_ref[...]


def _stage1(ref_feats, mask3, tgt_t):
    return pl.pallas_call(
        _stage1_body,
        grid=(NBLK,),
        in_specs=[
            pl.BlockSpec((BR, D), lambda i: (i, 0)),
            pl.BlockSpec((1, 1, BR), lambda i: (i, 0, 0)),
            pl.BlockSpec((N, D), lambda i: (0, 0)),
        ],
        out_specs=[
            pl.BlockSpec((1, 1, BR), lambda i: (i, 0, 0)),
            pl.BlockSpec((1, 1, BR), lambda i: (i, 0, 0)),
            pl.BlockSpec((1, N), lambda i: (0, 0)),
            pl.BlockSpec((1, N), lambda i: (0, 0)),
        ],
        out_shape=[
            jax.ShapeDtypeStruct((NBLK, 1, BR), jnp.int32),
            jax.ShapeDtypeStruct((NBLK, 1, BR), jnp.float32),
            jax.ShapeDtypeStruct((1, N), jnp.int32),
            jax.ShapeDtypeStruct((1, N), jnp.float32),
        ],
        scratch_shapes=[
            pltpu.VMEM((N, D), jnp.bfloat16),
            (pltpu.VMEM((D, N), jnp.bfloat16) if MATMUL_MODE == "bf16x3"
             else pltpu.VMEM((8, 128), jnp.float32) if MATMUL_MODE == "bf16"
             else pltpu.VMEM((D, N), jnp.float32)),
            pltpu.VMEM((1, N), jnp.float32),
            pltpu.VMEM((1, N), jnp.float32),
            pltpu.VMEM((1, N), jnp.float32),
        ],
        compiler_params=pltpu.CompilerParams(
            dimension_semantics=("arbitrary",),
        ),
    )(ref_feats, mask3, tgt_t)


NCH = N // 16


def _round_even(v):
    y = v + 0.5
    t = y.astype(jnp.int32)
    exact = t.astype(jnp.float32) == y
    odd = (t & 1) == 1
    t = jnp.where(exact & odd, t - 1, t)
    return t.astype(jnp.float32)


def _sc_body(fwt_hbm, fws_hbm, okt_hbm, csum_hbm, mask_hbm, osz_hbm, out_hbm,
             fwt_v, fws_v, okt_v, csum_v, mask_v, osz_v, sc2_v, cm_v, out_v):
    cid = lax.axis_index("c")
    sid = lax.axis_index("s")
    wid = sid * 2 + cid

    @pl.when(wid == 0)
    def _():
        pltpu.sync_copy(fwt_hbm, fwt_v)
        pltpu.sync_copy(fws_hbm, fws_v)
        pltpu.sync_copy(okt_hbm, okt_v)
        pltpu.sync_copy(csum_hbm, csum_v)
        pltpu.sync_copy(mask_hbm, mask_v)
        pltpu.sync_copy(osz_hbm, osz_v)

        io16 = lax.iota(jnp.int32, 16)
        negv = jnp.full((16,), NEG, jnp.float32)
        neg1 = jnp.full((16,), -1.0, jnp.float32)
        lowv = jnp.full((16,), -3e38, jnp.float32)

        ov = osz_v[pl.ds(0, 16)]
        zi = jnp.zeros((16,), jnp.int32)
        h_sc = jnp.max(jnp.where(io16 == 0, ov, zi))
        w_sc = jnp.max(jnp.where(io16 == 1, ov, zi))
        shv = jnp.full((16,), h_sc).astype(jnp.float32) / float(INPUT_SIZE)
        swv = jnp.full((16,), w_sc).astype(jnp.float32) / float(INPUT_SIZE)

        def _dsum(j, acc):
            b = j * 64
            return (acc + mask_v[pl.ds(b, 16)] + mask_v[pl.ds(b + 16, 16)]
                    + mask_v[pl.ds(b + 32, 16)] + mask_v[pl.ds(b + 48, 16)])
        msum = lax.fori_loop(0, NCH // 4, _dsum, jnp.zeros((16,), jnp.int32))
        denv = jnp.maximum(jnp.full((16,), jnp.sum(msum)).astype(jnp.float32),
                           1.0)

        def _oinit(j, c):
            out_v[pl.ds(j * 16, 16)] = neg1
            return c
        lax.fori_loop(0, (MAX_POINTS * 4) // 16, _oinit, 0)

        def _build(j, c):
            for u in range(4):
                jj = j * 4 + u
                tv = fwt_v[pl.ds(jj * 16, 16)]
                okv = plsc.load_gather(okt_v, [tv])
                mv = mask_v[pl.ds(jj * 16, 16)]
                sv = fws_v[pl.ds(jj * 16, 16)]
                sc = jnp.where((mv > 0) & (okv > 0), sv, negv)
                jv = jnp.full((16,), jj, jnp.int32)
                plsc.store_scatter(sc2_v, [jv, io16], sc)
                plsc.store_scatter(cm_v, [jv], jnp.full((16,), jnp.max(sc)))
            return c
        lax.fori_loop(0, NCH // 4, _build, 0)

        def _sup16():
            cm2 = lowv
            for s in range(16):
                ms = jnp.max(cm_v[pl.ds(s * 16, 16)])
                cm2 = jnp.where(io16 == s, jnp.full((16,), ms), cm2)
            return cm2

        def _extract(cm2):
            gmax = jnp.max(cm2)
            gs = jnp.full((16,), gmax)
            s = jnp.min(jnp.where(cm2 == gs, io16, 16))
            cmv = cm_v[pl.ds(s * 16, 16)]
            c16 = jnp.min(jnp.where(cmv == gs, io16, 16))
            c = s * 16 + c16
            cv = jnp.full((16,), c, jnp.int32)
            row = plsc.load_gather(sc2_v, [cv, io16])
            l = jnp.min(jnp.where(row == gs, io16, 16))
            idx = c * 16 + l
            nrow = jnp.where(io16 == l, lowv, row)
            plsc.store_scatter(sc2_v, [cv, io16], nrow)
            newc = jnp.full((16,), jnp.max(nrow))
            plsc.store_scatter(cm_v, [cv], newc)
            cmv2 = jnp.where(io16 == c16, newc, cmv)
            cm2 = jnp.where(io16 == s, jnp.full((16,), jnp.max(cmv2)), cm2)
            return idx, gs, cm2

        def _pack(m, xi, yi, scv, labv, acc):
            acc = jnp.where(io16 == 4 * m, xi, acc)
            acc = jnp.where(io16 == 4 * m + 1, yi, acc)
            acc = jnp.where(io16 == 4 * m + 2, scv, acc)
            acc = jnp.where(io16 == 4 * m + 3, labv, acc)
            return acc

        def _fg(k, carry):
            acc, cm2 = carry
            idx, gs, cm2 = _extract(cm2)
            tgt = plsc.load_gather(fwt_v, [jnp.full((16,), idx, jnp.int32)])
            okp = gs > (NEG / 2)
            scv = jnp.where(okp, gs, -1.0)
            labv = jnp.where(okp, 1.0, -1.0)
            xf = (tgt & (FEAT - 1)).astype(jnp.float32) * float(PATCH) + float(PATCH // 2)
            yf = lax.shift_right_logical(tgt, FEAT_SHIFT).astype(jnp.float32) * float(PATCH) + float(PATCH // 2)
            xi = jnp.where(okp, _round_even(xf * swv), -1.0)
            yi = jnp.where(okp, _round_even(yf * shv), -1.0)
            m = lax.rem(k, 4)
            acc = _pack(m, xi, yi, scv, labv, acc)

            @pl.when(m == 3)
            def _st():
                out_v[pl.ds((k // 4) * 16, 16)] = acc
            acc = jnp.where(jnp.full((16,), m, jnp.int32) == 3, neg1, acc)
            return acc, cm2
        lax.fori_loop(0, NUM_FG, _fg, (neg1, _sup16()))

        def _bbuild(j, c):
            for u in range(4):
                jj = j * 4 + u
                sc = -(csum_v[pl.ds(jj * 16, 16)] / denv)
                jv = jnp.full((16,), jj, jnp.int32)
                plsc.store_scatter(sc2_v, [jv, io16], sc)
                plsc.store_scatter(cm_v, [jv], jnp.full((16,), jnp.max(sc)))
            return c
        lax.fori_loop(0, NCH // 4, _bbuild, 0)

        def _bg(k, carry):
            acc, cm2 = carry
            idx, gs, cm2 = _extract(cm2)
            tgt = jnp.full((16,), idx, jnp.int32)
            xf = (tgt & (FEAT - 1)).astype(jnp.float32) * float(PATCH) + float(PATCH // 2)
            yf = lax.shift_right_logical(tgt, FEAT_SHIFT).astype(jnp.float32) * float(PATCH) + float(PATCH // 2)
            xi = _round_even(xf * swv)
            yi = _round_even(yf * shv)
            return _pack(k, xi, yi, -gs, jnp.zeros((16,), jnp.float32), acc), cm2
        acc2, _ = lax.fori_loop(0, NUM_BG, _bg, (neg1, _sup16()))
        out_v[pl.ds((NUM_FG // 4) * 16, 16)] = acc2

        pltpu.sync_copy(out_v, out_hbm)


def _stage2(fwt, fws, okt, csum, mask_i, osz16):
    mesh = plsc.VectorSubcoreMesh(core_axis_name="c", subcore_axis_name="s")
    f = functools.partial(
        pl.kernel,
        mesh=mesh,
        out_type=jax.ShapeDtypeStruct((MAX_POINTS * 4,), jnp.float32),
        scratch_types=[
            pltpu.VMEM((N,), jnp.int32),
            pltpu.VMEM((N,), jnp.float32),
            pltpu.VMEM((N,), jnp.int32),
            pltpu.VMEM((N,), jnp.float32),
            pltpu.VMEM((N,), jnp.int32),
            pltpu.VMEM((16,), jnp.int32),
            pltpu.VMEM((NCH, 16), jnp.float32),
            pltpu.VMEM((NCH,), jnp.float32),
            pltpu.VMEM((MAX_POINTS * 4,), jnp.float32),
        ],
        compiler_params=pltpu.CompilerParams(needs_layout_passes=False),
    )(_sc_body)
    return f(fwt, fws, okt, csum, mask_i, osz16)


def kernel(ref_feats, tgt_feats, ref_mask, original_size):
    mask_i = ref_mask.astype(jnp.int32)
    mask3 = mask_i.reshape(NBLK, 1, BR)
    tgt_t = tgt_feats
    fwidx, fwsc, okt, csum = _stage1(ref_feats, mask3, tgt_t)
    osz16 = jnp.zeros((16,), jnp.int32).at[:2].set(original_size)
    out = _stage2(fwidx.reshape(N), fwsc.reshape(N), okt.reshape(N),
                  csum.reshape(N), mask_i, osz16)
    return out.reshape(MAX_POINTS, 4)


# speedup vs baseline: 2.2173x; 1.0331x over previous
import functools

import jax
import jax.numpy as jnp
from jax import lax
from jax.experimental import pallas as pl
from jax.experimental.pallas import tpu as pltpu
from jax.experimental.pallas import tpu_sc as plsc

N = 4096
D = 768
BR = 1024
NBLK = N // BR
NEG = -1e9
FEAT = 64
FEAT_SHIFT = 6
PATCH = 16
INPUT_SIZE = 1024
NUM_FG = 40
NUM_BG = 2
MAX_POINTS = 64


MATMUL_MODE = "bf16"


def _stage1_body(ref_ref, mask_ref, tgtt_ref,
                 fwidx_ref, fwsc_ref, okt_ref, csum_ref,
                 tnh_ref, tnl_ref, cmax_all_ref, cmax_m_ref, csum_acc_ref):
    i = pl.program_id(0)

    @pl.when(i == 0)
    def _init():
        t = tgtt_ref[...]
        nrm = jnp.sqrt(jnp.sum(t * t, axis=1, keepdims=True))
        tn = t / (nrm + 1e-6)
        tnh = tn.astype(jnp.bfloat16)
        tnh_ref[...] = tnh
        if MATMUL_MODE == "bf16x3":
            tnl_ref[...] = (tn - tnh.astype(jnp.float32)).astype(jnp.bfloat16)
        elif MATMUL_MODE == "f32":
            tnl_ref[...] = tn
        cmax_all_ref[...] = jnp.full((1, N), -jnp.inf, jnp.float32)
        cmax_m_ref[...] = jnp.full((1, N), -jnp.inf, jnp.float32)
        csum_acc_ref[...] = jnp.zeros((1, N), jnp.float32)

    r = ref_ref[...]
    rn = r / (jnp.sqrt(jnp.sum(r * r, axis=1, keepdims=True)) + 1e-6)
    dotf = functools.partial(jnp.dot, preferred_element_type=jnp.float32)
    if MATMUL_MODE == "bf16":
        sim = lax.dot_general(rn.astype(jnp.bfloat16), tnh_ref[...],
                              dimension_numbers=(((1,), (1,)), ((), ())),
                              preferred_element_type=jnp.float32)
    elif MATMUL_MODE == "bf16x3":
        rh = rn.astype(jnp.bfloat16)
        rl = (rn - rh.astype(jnp.float32)).astype(jnp.bfloat16)
        sim = (dotf(rh, tnl_ref[...]) + dotf(rl, tnh_ref[...])
               + dotf(rh, tnh_ref[...]))
    else:
        sim = dotf(rn, tnl_ref[...])

    rowmax = jnp.max(sim, axis=1)
    colio = lax.broadcasted_iota(jnp.int32, (BR, N), 1)
    fwidx = jnp.min(jnp.where(sim == rowmax[:, None], colio, N), axis=1)
    fwidx_ref[0, 0, :] = fwidx
    fwsc_ref[0, 0, :] = rowmax

    m = mask_ref[0, 0, :]
    mcol = m[:, None] > 0
    bmax_all = jnp.max(sim, axis=0)
    bmax_m = jnp.max(jnp.where(mcol, sim, -jnp.inf), axis=0)
    bsum = jnp.sum(jnp.where(mcol, sim, 0.0), axis=0)
    cmax_all_ref[...] = jnp.maximum(cmax_all_ref[...], bmax_all[None, :])
    cmax_m_ref[...] = jnp.maximum(cmax_m_ref[...], bmax_m[None, :])
    csum_acc_ref[...] = csum_acc_ref[...] + bsum[None, :]

    @pl.when(i == NBLK - 1)
    def _fin():
        okt_ref[...] = (cmax_m_ref[...] == cmax_all_ref[...]).astype(jnp.int32)
        csum_ref[...] = csum_acc_ref[...]


def _stage1(ref_feats, mask3, tgt_t):
    return pl.pallas_call(
        _stage1_body,
        grid=(NBLK,),
        in_specs=[
            pl.BlockSpec((BR, D), lambda i: (i, 0)),
            pl.BlockSpec((1, 1, BR), lambda i: (i, 0, 0)),
            pl.BlockSpec((N, D), lambda i: (0, 0)),
        ],
        out_specs=[
            pl.BlockSpec((1, 1, BR), lambda i: (i, 0, 0)),
            pl.BlockSpec((1, 1, BR), lambda i: (i, 0, 0)),
            pl.BlockSpec((1, N), lambda i: (0, 0)),
            pl.BlockSpec((1, N), lambda i: (0, 0)),
        ],
        out_shape=[
            jax.ShapeDtypeStruct((NBLK, 1, BR), jnp.int32),
            jax.ShapeDtypeStruct((NBLK, 1, BR), jnp.float32),
            jax.ShapeDtypeStruct((1, N), jnp.int32),
            jax.ShapeDtypeStruct((1, N), jnp.float32),
        ],
        scratch_shapes=[
            pltpu.VMEM((N, D), jnp.bfloat16),
            (pltpu.VMEM((D, N), jnp.bfloat16) if MATMUL_MODE == "bf16x3"
             else pltpu.VMEM((8, 128), jnp.float32) if MATMUL_MODE == "bf16"
             else pltpu.VMEM((D, N), jnp.float32)),
            pltpu.VMEM((1, N), jnp.float32),
            pltpu.VMEM((1, N), jnp.float32),
            pltpu.VMEM((1, N), jnp.float32),
        ],
        compiler_params=pltpu.CompilerParams(
            dimension_semantics=("arbitrary",),
        ),
    )(ref_feats, mask3, tgt_t)


NCH = N // 16


def _round_even(v):
    y = v + 0.5
    t = y.astype(jnp.int32)
    exact = t.astype(jnp.float32) == y
    odd = (t & 1) == 1
    t = jnp.where(exact & odd, t - 1, t)
    return t.astype(jnp.float32)


def _sc_body(fwt_hbm, fws_hbm, okt_hbm, csum_hbm, mask_hbm, osz_hbm, out_hbm,
             fwt_v, fws_v, okt_v, csum_v, mask_v, osz_v, sc2_v, cm_v, out_v,
             sem):
    cid = lax.axis_index("c")
    sid = lax.axis_index("s")
    wid = sid * 2 + cid

    @pl.when(wid == 0)
    def _():
        cps = [pltpu.make_async_copy(src, dst, sem)
               for src, dst in ((fwt_hbm, fwt_v), (fws_hbm, fws_v),
                                (okt_hbm, okt_v), (csum_hbm, csum_v),
                                (mask_hbm, mask_v), (osz_hbm, osz_v))]
        for cp in cps:
            cp.start()
        for cp in cps:
            cp.wait()

        io16 = lax.iota(jnp.int32, 16)
        negv = jnp.full((16,), NEG, jnp.float32)
        neg1 = jnp.full((16,), -1.0, jnp.float32)
        lowv = jnp.full((16,), -3e38, jnp.float32)

        ov = osz_v[pl.ds(0, 16)]
        zi = jnp.zeros((16,), jnp.int32)
        h_sc = jnp.max(jnp.where(io16 == 0, ov, zi))
        w_sc = jnp.max(jnp.where(io16 == 1, ov, zi))
        shv = jnp.full((16,), h_sc).astype(jnp.float32) / float(INPUT_SIZE)
        swv = jnp.full((16,), w_sc).astype(jnp.float32) / float(INPUT_SIZE)

        def _dsum(j, acc):
            b = j * 64
            return (acc + mask_v[pl.ds(b, 16)] + mask_v[pl.ds(b + 16, 16)]
                    + mask_v[pl.ds(b + 32, 16)] + mask_v[pl.ds(b + 48, 16)])
        msum = lax.fori_loop(0, NCH // 4, _dsum, jnp.zeros((16,), jnp.int32))
        denv = jnp.maximum(jnp.full((16,), jnp.sum(msum)).astype(jnp.float32),
                           1.0)

        def _oinit(j, c):
            out_v[pl.ds(j * 16, 16)] = neg1
            return c
        lax.fori_loop(0, (MAX_POINTS * 4) // 16, _oinit, 0)

        def _build(j, c):
            for u in range(4):
                jj = j * 4 + u
                tv = fwt_v[pl.ds(jj * 16, 16)]
                okv = plsc.load_gather(okt_v, [tv])
                mv = mask_v[pl.ds(jj * 16, 16)]
                sv = fws_v[pl.ds(jj * 16, 16)]
                sc = jnp.where((mv > 0) & (okv > 0), sv, negv)
                jv = jnp.full((16,), jj, jnp.int32)
                plsc.store_scatter(sc2_v, [jv, io16], sc)
                plsc.store_scatter(cm_v, [jv], jnp.full((16,), jnp.max(sc)))
            return c
        lax.fori_loop(0, NCH // 4, _build, 0)

        def _sup16():
            cm2 = lowv
            for s in range(16):
                ms = jnp.max(cm_v[pl.ds(s * 16, 16)])
                cm2 = jnp.where(io16 == s, jnp.full((16,), ms), cm2)
            return cm2

        def _extract(cm2):
            gmax = jnp.max(cm2)
            gs = jnp.full((16,), gmax)
            s = jnp.min(jnp.where(cm2 == gs, io16, 16))
            cmv = cm_v[pl.ds(s * 16, 16)]
            c16 = jnp.min(jnp.where(cmv == gs, io16, 16))
            c = s * 16 + c16
            cv = jnp.full((16,), c, jnp.int32)
            row = plsc.load_gather(sc2_v, [cv, io16])
            l = jnp.min(jnp.where(row == gs, io16, 16))
            idx = c * 16 + l
            nrow = jnp.where(io16 == l, lowv, row)
            plsc.store_scatter(sc2_v, [cv, io16], nrow)
            newc = jnp.full((16,), jnp.max(nrow))
            plsc.store_scatter(cm_v, [cv], newc)
            cmv2 = jnp.where(io16 == c16, newc, cmv)
            cm2 = jnp.where(io16 == s, jnp.full((16,), jnp.max(cmv2)), cm2)
            return idx, gs, cm2

        def _pack(m, xi, yi, scv, labv, acc):
            acc = jnp.where(io16 == 4 * m, xi, acc)
            acc = jnp.where(io16 == 4 * m + 1, yi, acc)
            acc = jnp.where(io16 == 4 * m + 2, scv, acc)
            acc = jnp.where(io16 == 4 * m + 3, labv, acc)
            return acc

        def _fg(k, carry):
            acc, cm2 = carry
            idx, gs, cm2 = _extract(cm2)
            tgt = plsc.load_gather(fwt_v, [jnp.full((16,), idx, jnp.int32)])
            okp = gs > (NEG / 2)
            scv = jnp.where(okp, gs, -1.0)
            labv = jnp.where(okp, 1.0, -1.0)
            xf = (tgt & (FEAT - 1)).astype(jnp.float32) * float(PATCH) + float(PATCH // 2)
            yf = lax.shift_right_logical(tgt, FEAT_SHIFT).astype(jnp.float32) * float(PATCH) + float(PATCH // 2)
            xi = jnp.where(okp, _round_even(xf * swv), -1.0)
            yi = jnp.where(okp, _round_even(yf * shv), -1.0)
            m = lax.rem(k, 4)
            acc = _pack(m, xi, yi, scv, labv, acc)

            @pl.when(m == 3)
            def _st():
                out_v[pl.ds((k // 4) * 16, 16)] = acc
            acc = jnp.where(jnp.full((16,), m, jnp.int32) == 3, neg1, acc)
            return acc, cm2
        lax.fori_loop(0, NUM_FG, _fg, (neg1, _sup16()))

        def _bbuild(j, c):
            for u in range(4):
                jj = j * 4 + u
                sc = -(csum_v[pl.ds(jj * 16, 16)] / denv)
                jv = jnp.full((16,), jj, jnp.int32)
                plsc.store_scatter(sc2_v, [jv, io16], sc)
                plsc.store_scatter(cm_v, [jv], jnp.full((16,), jnp.max(sc)))
            return c
        lax.fori_loop(0, NCH // 4, _bbuild, 0)

        def _bg(k, carry):
            acc, cm2 = carry
            idx, gs, cm2 = _extract(cm2)
            tgt = jnp.full((16,), idx, jnp.int32)
            xf = (tgt & (FEAT - 1)).astype(jnp.float32) * float(PATCH) + float(PATCH // 2)
            yf = lax.shift_right_logical(tgt, FEAT_SHIFT).astype(jnp.float32) * float(PATCH) + float(PATCH // 2)
            xi = _round_even(xf * swv)
            yi = _round_even(yf * shv)
            return _pack(k, xi, yi, -gs, jnp.zeros((16,), jnp.float32), acc), cm2
        acc2, _ = lax.fori_loop(0, NUM_BG, _bg, (neg1, _sup16()))
        out_v[pl.ds((NUM_FG // 4) * 16, 16)] = acc2

        pltpu.sync_copy(out_v, out_hbm)


def _stage2(fwt, fws, okt, csum, mask_i, osz16):
    mesh = plsc.VectorSubcoreMesh(core_axis_name="c", subcore_axis_name="s")
    f = functools.partial(
        pl.kernel,
        mesh=mesh,
        out_type=jax.ShapeDtypeStruct((MAX_POINTS * 4,), jnp.float32),
        scratch_types=[
            pltpu.VMEM((N,), jnp.int32),
            pltpu.VMEM((N,), jnp.float32),
            pltpu.VMEM((N,), jnp.int32),
            pltpu.VMEM((N,), jnp.float32),
            pltpu.VMEM((N,), jnp.int32),
            pltpu.VMEM((16,), jnp.int32),
            pltpu.VMEM((NCH, 16), jnp.float32),
            pltpu.VMEM((NCH,), jnp.float32),
            pltpu.VMEM((MAX_POINTS * 4,), jnp.float32),
            pltpu.SemaphoreType.DMA,
        ],
        compiler_params=pltpu.CompilerParams(needs_layout_passes=False),
    )(_sc_body)
    return f(fwt, fws, okt, csum, mask_i, osz16)


def kernel(ref_feats, tgt_feats, ref_mask, original_size):
    mask_i = ref_mask.astype(jnp.int32)
    mask3 = mask_i.reshape(NBLK, 1, BR)
    tgt_t = tgt_feats
    fwidx, fwsc, okt, csum = _stage1(ref_feats, mask3, tgt_t)
    osz16 = jnp.zeros((16,), jnp.int32).at[:2].set(original_size)
    out = _stage2(fwidx.reshape(N), fwsc.reshape(N), okt.reshape(N),
                  csum.reshape(N), mask_i, osz16)
    return out.reshape(MAX_POINTS, 4)


# speedup vs baseline: 2.2198x; 1.0011x over previous
import functools

import jax
import jax.numpy as jnp
from jax import lax
from jax.experimental import pallas as pl
from jax.experimental.pallas import tpu as pltpu
from jax.experimental.pallas import tpu_sc as plsc

N = 4096
D = 768
BR = 1024
NBLK = N // BR
NEG = -1e9
FEAT = 64
FEAT_SHIFT = 6
PATCH = 16
INPUT_SIZE = 1024
NUM_FG = 40
NUM_BG = 2
MAX_POINTS = 64


def _stage1_body(ref_ref, mask_ref, tgtt_ref,
                 fwidx_ref, fwsc_ref, okt_ref, csum_ref,
                 tnh_ref, cmax_all_ref, cmax_m_ref, csum_acc_ref):
    i = pl.program_id(0)

    @pl.when(i == 0)
    def _init():
        t = tgtt_ref[...]
        nrm = jnp.sqrt(jnp.sum(t * t, axis=1, keepdims=True))
        tn = t / (nrm + 1e-6)
        tnh_ref[...] = tn.astype(jnp.bfloat16)
        cmax_all_ref[...] = jnp.full((1, N), -jnp.inf, jnp.float32)
        cmax_m_ref[...] = jnp.full((1, N), -jnp.inf, jnp.float32)
        csum_acc_ref[...] = jnp.zeros((1, N), jnp.float32)

    r = ref_ref[...]
    rn = r / (jnp.sqrt(jnp.sum(r * r, axis=1, keepdims=True)) + 1e-6)
    sim = lax.dot_general(rn.astype(jnp.bfloat16), tnh_ref[...],
                          dimension_numbers=(((1,), (1,)), ((), ())),
                          preferred_element_type=jnp.float32)

    rowmax = jnp.max(sim, axis=1)
    colio = lax.broadcasted_iota(jnp.int32, (BR, N), 1)
    fwidx = jnp.min(jnp.where(sim == rowmax[:, None], colio, N), axis=1)
    fwidx_ref[0, 0, :] = fwidx
    fwsc_ref[0, 0, :] = rowmax

    m = mask_ref[0, 0, :]
    mcol = m[:, None] > 0
    bmax_all = jnp.max(sim, axis=0)
    bmax_m = jnp.max(jnp.where(mcol, sim, -jnp.inf), axis=0)
    bsum = jnp.sum(jnp.where(mcol, sim, 0.0), axis=0)
    cmax_all_ref[...] = jnp.maximum(cmax_all_ref[...], bmax_all[None, :])
    cmax_m_ref[...] = jnp.maximum(cmax_m_ref[...], bmax_m[None, :])
    csum_acc_ref[...] = csum_acc_ref[...] + bsum[None, :]

    @pl.when(i == NBLK - 1)
    def _fin():
        okt_ref[...] = (cmax_m_ref[...] == cmax_all_ref[...]).astype(jnp.int32)
        csum_ref[...] = csum_acc_ref[...]


def _stage1(ref_feats, mask3, tgt_t):
    return pl.pallas_call(
        _stage1_body,
        grid=(NBLK,),
        in_specs=[
            pl.BlockSpec((BR, D), lambda i: (i, 0)),
            pl.BlockSpec((1, 1, BR), lambda i: (i, 0, 0)),
            pl.BlockSpec((N, D), lambda i: (0, 0)),
        ],
        out_specs=[
            pl.BlockSpec((1, 1, BR), lambda i: (i, 0, 0)),
            pl.BlockSpec((1, 1, BR), lambda i: (i, 0, 0)),
            pl.BlockSpec((1, N), lambda i: (0, 0)),
            pl.BlockSpec((1, N), lambda i: (0, 0)),
        ],
        out_shape=[
            jax.ShapeDtypeStruct((NBLK, 1, BR), jnp.int32),
            jax.ShapeDtypeStruct((NBLK, 1, BR), jnp.float32),
            jax.ShapeDtypeStruct((1, N), jnp.int32),
            jax.ShapeDtypeStruct((1, N), jnp.float32),
        ],
        scratch_shapes=[
            pltpu.VMEM((N, D), jnp.bfloat16),
            pltpu.VMEM((1, N), jnp.float32),
            pltpu.VMEM((1, N), jnp.float32),
            pltpu.VMEM((1, N), jnp.float32),
        ],
        compiler_params=pltpu.CompilerParams(
            dimension_semantics=("arbitrary",),
        ),
    )(ref_feats, mask3, tgt_t)


NCH = N // 16


def _round_even(v):
    y = v + 0.5
    t = y.astype(jnp.int32)
    exact = t.astype(jnp.float32) == y
    odd = (t & 1) == 1
    t = jnp.where(exact & odd, t - 1, t)
    return t.astype(jnp.float32)


def _sc_body(fwt_hbm, fws_hbm, okt_hbm, csum_hbm, mask_hbm, osz_hbm, out_hbm,
             fwt_v, fws_v, okt_v, csum_v, mask_v, osz_v, sc2_v, cm_v, out_v,
             sem):
    cid = lax.axis_index("c")
    sid = lax.axis_index("s")
    wid = sid * 2 + cid

    @pl.when(wid == 0)
    def _():
        cps = [pltpu.make_async_copy(src, dst, sem)
               for src, dst in ((fwt_hbm, fwt_v), (fws_hbm, fws_v),
                                (okt_hbm, okt_v), (csum_hbm, csum_v),
                                (mask_hbm, mask_v), (osz_hbm, osz_v))]
        for cp in cps:
            cp.start()
        for cp in cps:
            cp.wait()

        io16 = lax.iota(jnp.int32, 16)
        negv = jnp.full((16,), NEG, jnp.float32)
        neg1 = jnp.full((16,), -1.0, jnp.float32)
        lowv = jnp.full((16,), -3e38, jnp.float32)

        ov = osz_v[pl.ds(0, 16)]
        zi = jnp.zeros((16,), jnp.int32)
        h_sc = jnp.max(jnp.where(io16 == 0, ov, zi))
        w_sc = jnp.max(jnp.where(io16 == 1, ov, zi))
        shv = jnp.full((16,), h_sc).astype(jnp.float32) / float(INPUT_SIZE)
        swv = jnp.full((16,), w_sc).astype(jnp.float32) / float(INPUT_SIZE)

        def _dsum(j, acc):
            b = j * 64
            return (acc + mask_v[pl.ds(b, 16)] + mask_v[pl.ds(b + 16, 16)]
                    + mask_v[pl.ds(b + 32, 16)] + mask_v[pl.ds(b + 48, 16)])
        msum = lax.fori_loop(0, NCH // 4, _dsum, jnp.zeros((16,), jnp.int32))
        denv = jnp.maximum(jnp.full((16,), jnp.sum(msum)).astype(jnp.float32),
                           1.0)

        def _oinit(j, c):
            out_v[pl.ds(j * 16, 16)] = neg1
            return c
        lax.fori_loop(0, (MAX_POINTS * 4) // 16, _oinit, 0)

        def _build(j, c):
            for u in range(4):
                jj = j * 4 + u
                tv = fwt_v[pl.ds(jj * 16, 16)]
                okv = plsc.load_gather(okt_v, [tv])
                mv = mask_v[pl.ds(jj * 16, 16)]
                sv = fws_v[pl.ds(jj * 16, 16)]
                sc = jnp.where((mv > 0) & (okv > 0), sv, negv)
                jv = jnp.full((16,), jj, jnp.int32)
                plsc.store_scatter(sc2_v, [jv, io16], sc)
                plsc.store_scatter(cm_v, [jv], jnp.full((16,), jnp.max(sc)))
            return c
        lax.fori_loop(0, NCH // 4, _build, 0)

        def _sup16():
            cm2 = lowv
            for s in range(16):
                ms = jnp.max(cm_v[pl.ds(s * 16, 16)])
                cm2 = jnp.where(io16 == s, jnp.full((16,), ms), cm2)
            return cm2

        def _extract(cm2):
            gmax = jnp.max(cm2)
            gs = jnp.full((16,), gmax)
            s = jnp.min(jnp.where(cm2 == gs, io16, 16))
            cmv = cm_v[pl.ds(s * 16, 16)]
            c16 = jnp.min(jnp.where(cmv == gs, io16, 16))
            c = s * 16 + c16
            cv = jnp.full((16,), c, jnp.int32)
            row = plsc.load_gather(sc2_v, [cv, io16])
            l = jnp.min(jnp.where(row == gs, io16, 16))
            idx = c * 16 + l
            nrow = jnp.where(io16 == l, lowv, row)
            plsc.store_scatter(sc2_v, [cv, io16], nrow)
            newc = jnp.full((16,), jnp.max(nrow))
            plsc.store_scatter(cm_v, [cv], newc)
            cmv2 = jnp.where(io16 == c16, newc, cmv)
            cm2 = jnp.where(io16 == s, jnp.full((16,), jnp.max(cmv2)), cm2)
            return idx, gs, cm2

        def _pack(m, xi, yi, scv, labv, acc):
            acc = jnp.where(io16 == 4 * m, xi, acc)
            acc = jnp.where(io16 == 4 * m + 1, yi, acc)
            acc = jnp.where(io16 == 4 * m + 2, scv, acc)
            acc = jnp.where(io16 == 4 * m + 3, labv, acc)
            return acc

        def _fg(k, carry):
            acc, cm2 = carry
            idx, gs, cm2 = _extract(cm2)
            tgt = plsc.load_gather(fwt_v, [jnp.full((16,), idx, jnp.int32)])
            okp = gs > (NEG / 2)
            scv = jnp.where(okp, gs, -1.0)
            labv = jnp.where(okp, 1.0, -1.0)
            xf = (tgt & (FEAT - 1)).astype(jnp.float32) * float(PATCH) + float(PATCH // 2)
            yf = lax.shift_right_logical(tgt, FEAT_SHIFT).astype(jnp.float32) * float(PATCH) + float(PATCH // 2)
            xi = jnp.where(okp, _round_even(xf * swv), -1.0)
            yi = jnp.where(okp, _round_even(yf * shv), -1.0)
            m = lax.rem(k, 4)
            acc = _pack(m, xi, yi, scv, labv, acc)

            @pl.when(m == 3)
            def _st():
                out_v[pl.ds((k // 4) * 16, 16)] = acc
            acc = jnp.where(jnp.full((16,), m, jnp.int32) == 3, neg1, acc)
            return acc, cm2
        lax.fori_loop(0, NUM_FG, _fg, (neg1, _sup16()))

        def _bbuild(j, c):
            for u in range(4):
                jj = j * 4 + u
                sc = -(csum_v[pl.ds(jj * 16, 16)] / denv)
                jv = jnp.full((16,), jj, jnp.int32)
                plsc.store_scatter(sc2_v, [jv, io16], sc)
                plsc.store_scatter(cm_v, [jv], jnp.full((16,), jnp.max(sc)))
            return c
        lax.fori_loop(0, NCH // 4, _bbuild, 0)

        def _bg(k, carry):
            acc, cm2 = carry
            idx, gs, cm2 = _extract(cm2)
            tgt = jnp.full((16,), idx, jnp.int32)
            xf = (tgt & (FEAT - 1)).astype(jnp.float32) * float(PATCH) + float(PATCH // 2)
            yf = lax.shift_right_logical(tgt, FEAT_SHIFT).astype(jnp.float32) * float(PATCH) + float(PATCH // 2)
            xi = _round_even(xf * swv)
            yi = _round_even(yf * shv)
            return _pack(k, xi, yi, -gs, jnp.zeros((16,), jnp.float32), acc), cm2
        acc2, _ = lax.fori_loop(0, NUM_BG, _bg, (neg1, _sup16()))
        out_v[pl.ds((NUM_FG // 4) * 16, 16)] = acc2

        pltpu.sync_copy(out_v, out_hbm)


def _stage2(fwt, fws, okt, csum, mask_i, osz16):
    mesh = plsc.VectorSubcoreMesh(core_axis_name="c", subcore_axis_name="s")
    f = functools.partial(
        pl.kernel,
        mesh=mesh,
        out_type=jax.ShapeDtypeStruct((MAX_POINTS * 4,), jnp.float32),
        scratch_types=[
            pltpu.VMEM((N,), jnp.int32),
            pltpu.VMEM((N,), jnp.float32),
            pltpu.VMEM((N,), jnp.int32),
            pltpu.VMEM((N,), jnp.float32),
            pltpu.VMEM((N,), jnp.int32),
            pltpu.VMEM((16,), jnp.int32),
            pltpu.VMEM((NCH, 16), jnp.float32),
            pltpu.VMEM((NCH,), jnp.float32),
            pltpu.VMEM((MAX_POINTS * 4,), jnp.float32),
            pltpu.SemaphoreType.DMA,
        ],
        compiler_params=pltpu.CompilerParams(needs_layout_passes=False),
    )(_sc_body)
    return f(fwt, fws, okt, csum, mask_i, osz16)


def kernel(ref_feats, tgt_feats, ref_mask, original_size):
    mask_i = ref_mask.astype(jnp.int32)
    mask3 = mask_i.reshape(NBLK, 1, BR)
    tgt_t = tgt_feats
    fwidx, fwsc, okt, csum = _stage1(ref_feats, mask3, tgt_t)
    osz16 = jnp.zeros((16,), jnp.int32).at[:2].set(original_size)
    out = _stage2(fwidx.reshape(N), fwsc.reshape(N), okt.reshape(N),
                  csum.reshape(N), mask_i, osz16)
    return out.reshape(MAX_POINTS, 4)
